# Initial kernel scaffold; baseline (speedup 1.0000x reference)
import jax
import jax.numpy as jnp
from jax.experimental import pallas as pl


def kernel(pred, target, batch, x):
    raise NotImplementedError("write your pallas kernel here")


# speedup vs baseline: 11.3973x; 11.3973x over previous
import jax
import jax.numpy as jnp
from jax.experimental import pallas as pl

_EPS = 0.1
_NUM_GRAPHS = 64
_LANES = 128


def _grel_kernel(pred_ref, targ_ref, batch_ref, out_ref, *, last_row, last_col):
    p = pred_ref[...]
    t = targ_ref[...]
    b = batch_ref[...]
    rel = jnp.abs(p - t) / (jnp.abs(t) + _EPS)
    num_graphs = batch_ref[last_row, last_col] + 1

    def body(g, total):
        mask = b == g
        s = jnp.sum(jnp.where(mask, rel, 0.0))
        c = jnp.sum(jnp.where(mask, 1.0, 0.0))
        mean_g = s / c
        return total + jnp.where(g < num_graphs, mean_g, 0.0)

    total = jax.lax.fori_loop(0, _NUM_GRAPHS, body, jnp.float32(0.0))
    result = total / num_graphs.astype(jnp.float32) * 10000.0
    out_ref[...] = jnp.broadcast_to(result, (1, 1))


def kernel(pred, target, batch, x):
    del x
    n = pred.shape[0]
    rows = (n + _LANES - 1) // _LANES
    rows = ((rows + 7) // 8) * 8
    padded = rows * _LANES
    pad = padded - n
    batch = batch.astype(jnp.int32)
    pred2 = jnp.pad(pred, (0, pad)).reshape(rows, _LANES)
    targ2 = jnp.pad(target, (0, pad)).reshape(rows, _LANES)
    batch2 = jnp.pad(batch, (0, pad), constant_values=_NUM_GRAPHS).reshape(
        rows, _LANES
    )
    import functools

    out = pl.pallas_call(
        functools.partial(
            _grel_kernel,
            last_row=(n - 1) // _LANES,
            last_col=(n - 1) % _LANES,
        ),
        out_shape=jax.ShapeDtypeStruct((1, 1), jnp.float32),
    )(pred2, targ2, batch2)
    return out.reshape(())
